# baseline (speedup 1.0000x reference)
import jax
import jax.numpy as jnp
from jax.experimental import pallas as pl

N_CLS = 20


def _body(x_ref, o_ref):
    lab = x_ref[0, 0]
    classes = jax.lax.broadcasted_iota(jnp.int32, (N_CLS,) + lab.shape, 0)
    o_ref[0] = (lab[None, :, :] == classes).astype(jnp.float32)


def kernel(x):
    B, _, H, W = x.shape
    HB = 128
    out = pl.pallas_call(
        _body,
        grid=(B, H // HB),
        in_specs=[pl.BlockSpec((1, 1, HB, W), lambda b, h: (b, 0, h, 0))],
        out_specs=pl.BlockSpec((1, N_CLS, HB, W), lambda b, h: (b, 0, h, 0)),
        out_shape=jax.ShapeDtypeStruct((B, N_CLS, H, W), jnp.float32),
    )(x)
    return out
